# baseline (speedup 1.0000x reference)
import functools

import jax
import jax.numpy as jnp
from jax import lax
from jax.experimental import pallas as pl
from jax.experimental.pallas import tpu as pltpu
from jax.experimental.pallas import tpu_sc as plsc

NW = 32
BT = 128
NBUF = 4


def _gather_kernel(per_w, H, x_hbm, table_hbm, out_hbm,
                   idx_v, idx_t, rows_v, trans_v, gsems, osems):
    nc = 2
    wid = lax.axis_index("s") * nc + lax.axis_index("c")
    s0 = wid * per_w
    ntile = per_w // BT
    nblk = ntile * H
    lane = lax.iota(jnp.int32, 16)
    rows16 = [c * 16 + lane for c in range(BT // 16)]

    pltpu.sync_copy(x_hbm.at[pl.ds(s0, per_w)], idx_v)

    rows_pw = [c * 16 + lane for c in range(per_w // 16)]

    @pl.loop(0, H)
    def _(h):
        cols = lane * 0 + h
        for chunk in range(per_w // 16):
            v = plsc.load_gather(idx_v, [rows_pw[chunk], cols])
            idx_t[h, pl.ds(chunk * 16, 16)] = v

    @pl.loop(0, nblk, step=NBUF)
    def group(i0):
        descs = []
        for b in range(NBUF):
            i = i0 + b
            h = lax.rem(i, H)
            bsub = lax.div(i, H)
            @pl.when(i0 > 0)
            def _(b=b):
                pltpu.make_async_copy(
                    trans_v.at[b], out_hbm.at[0, :, 0], osems[b]
                ).wait()
            descs.append(
                pltpu.async_copy(
                    table_hbm.at[idx_t.at[h, pl.ds(bsub * BT, BT)]],
                    rows_v.at[b], gsems[b],
                )
            )
        for b in range(NBUF):
            i = i0 + b
            h = lax.rem(i, H)
            bsub = lax.div(i, H)
            bj = wid * ntile + bsub
            descs[b].wait()

            @pl.loop(0, 8)
            def _(ti, b=b):
                d0 = ti * 8
                for di in range(8):
                    cols = lane * 0 + (d0 + di)
                    for chunk in range(BT // 16):
                        v = plsc.load_gather(
                            rows_v.at[b], [rows16[chunk], cols])
                        trans_v[b, ti, di, pl.ds(chunk * 16, 16)] = v

            pltpu.async_copy(
                trans_v.at[b], out_hbm.at[h, :, bj], osems[b]
            )

    for b in range(NBUF):
        pltpu.make_async_copy(
            trans_v.at[b], out_hbm.at[0, :, 0], osems[b]
        ).wait()


def kernel(x, table):
    B, H = x.shape
    V, D = table.shape
    per_w = B // NW
    assert per_w * NW == B and per_w % BT == 0 and D == 64 and BT == 128

    mesh = plsc.VectorSubcoreMesh(core_axis_name="c", subcore_axis_name="s")
    run = pl.kernel(
        functools.partial(_gather_kernel, per_w, H),
        out_type=jax.ShapeDtypeStruct((H, D // 8, B // BT, 8, BT), jnp.float32),
        mesh=mesh,
        scratch_types=[
            pltpu.VMEM((per_w, H), jnp.int32),
            pltpu.VMEM((H, per_w), jnp.int32),
            pltpu.VMEM((NBUF, BT, D), jnp.float32),
            pltpu.VMEM((NBUF, D // 8, 8, BT), jnp.float32),
            [pltpu.SemaphoreType.DMA] * NBUF,
            [pltpu.SemaphoreType.DMA] * NBUF,
        ],
        compiler_params=pltpu.CompilerParams(use_tc_tiling_on_sc=False, needs_layout_passes=False),
    )
    out5 = run(x.astype(jnp.int32), table)
    return out5.transpose(2, 4, 0, 1, 3).reshape(B, H, D)


# speedup vs baseline: 1.1651x; 1.1651x over previous
import functools

import jax
import jax.numpy as jnp
from jax import lax
from jax.experimental import pallas as pl
from jax.experimental.pallas import tpu as pltpu
from jax.experimental.pallas import tpu_sc as plsc

NW = 32
BT = 128
NBUF = 4


def _gather_kernel(per_w, H, x_hbm, table_hbm, out_hbm,
                   idx_v, idx_t, rows_v, trans_v, gsems, osems):
    nc = 2
    wid = lax.axis_index("s") * nc + lax.axis_index("c")
    s0 = wid * per_w
    ntile = per_w // BT
    nblk = ntile * H
    D = 64
    lane = lax.iota(jnp.int32, 16)
    rows16 = [c * 16 + lane for c in range(per_w // 16)]

    pltpu.sync_copy(x_hbm.at[pl.ds(s0, per_w)], idx_v)

    @pl.loop(0, H)
    def _(h):
        cols = lane * 0 + h
        for c0 in range(0, per_w // 16, 8):
            vs = [plsc.load_gather(idx_v, [rows16[c], cols])
                  for c in range(c0, c0 + 8)]
            for j in range(8):
                idx_t[h, pl.ds((c0 + j) * 16, 16)] = vs[j]

    @pl.loop(0, nblk, step=NBUF)
    def group(i0):
        descs = []
        for b in range(NBUF):
            i = i0 + b
            h = lax.rem(i, H)
            bsub = lax.div(i, H)
            @pl.when(i0 > 0)
            def _(b=b):
                pltpu.make_async_copy(
                    trans_v.at[b], out_hbm.at[0, :, 0], osems[b]
                ).wait()
            descs.append(
                pltpu.async_copy(
                    table_hbm.at[idx_t.at[h, pl.ds(bsub * BT, BT)]],
                    rows_v.at[b], gsems[b],
                )
            )
        for b in range(NBUF):
            i = i0 + b
            h = lax.rem(i, H)
            bsub = lax.div(i, H)
            bj = wid * ntile + bsub
            descs[b].wait()

            @pl.loop(0, D // 8)
            def _(ti, b=b):
                for di in range(8):
                    cols = lane * 0 + (ti * 8 + di)
                    vs = [plsc.load_gather(rows_v.at[b], [rows16[c], cols])
                          for c in range(BT // 16)]
                    for c in range(BT // 16):
                        trans_v[b, ti, di, pl.ds(c * 16, 16)] = vs[c]

            pltpu.async_copy(
                trans_v.at[b], out_hbm.at[h, :, bj], osems[b]
            )

    for b in range(NBUF):
        pltpu.make_async_copy(
            trans_v.at[b], out_hbm.at[0, :, 0], osems[b]
        ).wait()


def kernel(x, table):
    B, H = x.shape
    V, D = table.shape
    per_w = B // NW
    assert per_w * NW == B and per_w % BT == 0 and D == 64 and BT == 128

    mesh = plsc.VectorSubcoreMesh(core_axis_name="c", subcore_axis_name="s")
    run = pl.kernel(
        functools.partial(_gather_kernel, per_w, H),
        out_type=jax.ShapeDtypeStruct((H, D // 8, B // BT, 8, BT), jnp.float32),
        mesh=mesh,
        scratch_types=[
            pltpu.VMEM((per_w, H), jnp.int32),
            pltpu.VMEM((H, per_w), jnp.int32),
            pltpu.VMEM((NBUF, BT, D), jnp.float32),
            pltpu.VMEM((NBUF, D // 8, 8, BT), jnp.float32),
            [pltpu.SemaphoreType.DMA] * NBUF,
            [pltpu.SemaphoreType.DMA] * NBUF,
        ],
        compiler_params=pltpu.CompilerParams(
            use_tc_tiling_on_sc=False, needs_layout_passes=False),
    )
    out5 = run(x.astype(jnp.int32), table)
    return out5.transpose(2, 4, 0, 1, 3).reshape(B, H, D)


# speedup vs baseline: 1.4678x; 1.2598x over previous
import functools

import jax
import jax.numpy as jnp
from jax import lax
from jax.experimental import pallas as pl
from jax.experimental.pallas import tpu as pltpu
from jax.experimental.pallas import tpu_sc as plsc

NW = 32
BT = 128
NBUF = 4


def _gather_kernel(per_w, H, D, x_hbm, table_hbm, out_hbm,
                   idx_v, idx_t, rows_v, stage_v, trans_v, gsems, osems):
    nc = 2
    wid = lax.axis_index("s") * nc + lax.axis_index("c")
    s0 = wid * per_w
    ntile = per_w // BT
    nblk = ntile * H
    lane = lax.iota(jnp.int32, 16)
    rows16 = [c * 16 + lane for c in range(per_w // 16)]

    pltpu.sync_copy(x_hbm.at[pl.ds(s0, per_w)], idx_v)

    @pl.loop(0, H)
    def _(h):
        cols = lane * 0 + h
        for c0 in range(0, per_w // 16, 8):
            vs = [plsc.load_gather(idx_v, [rows16[c], cols])
                  for c in range(c0, c0 + 8)]
            for j in range(8):
                idx_t[h, pl.ds((c0 + j) * 16, 16)] = vs[j]

    @pl.loop(0, nblk, step=NBUF)
    def group(i0):
        descs = []
        for b in range(NBUF):
            i = i0 + b
            h = lax.rem(i, H)
            bsub = lax.div(i, H)
            @pl.when(i0 > 0)
            def _(b=b):
                pltpu.make_async_copy(
                    trans_v.at[b], out_hbm.at[0, :, 0], osems[b]
                ).wait()
            descs.append(
                pltpu.async_copy(
                    table_hbm.at[idx_t.at[h, pl.ds(bsub * BT, BT)]],
                    rows_v.at[b], gsems[b],
                )
            )
        for b in range(NBUF):
            i = i0 + b
            h = lax.rem(i, H)
            bsub = lax.div(i, H)
            bj = wid * ntile + bsub
            descs[b].wait()
            @pl.loop(0, BT, step=4)
            def _(bi0, b=b):
                for u in range(4):
                    for c in range(D // 16):
                        stage_v[bi0 + u, pl.ds(c * 16, 16)] = (
                            rows_v[b, bi0 + u, pl.ds(c * 16, 16)])

            @pl.loop(0, D // 8)
            def _(ti, b=b):
                for di in range(8):
                    cols = lane * 0 + (ti * 8 + di)
                    vs = [plsc.load_gather(stage_v, [rows16[c], cols])
                          for c in range(BT // 16)]
                    for c in range(BT // 16):
                        trans_v[b, ti, di, pl.ds(c * 16, 16)] = vs[c]

            pltpu.async_copy(
                trans_v.at[b], out_hbm.at[h, :, bj], osems[b]
            )

    for b in range(NBUF):
        pltpu.make_async_copy(
            trans_v.at[b], out_hbm.at[0, :, 0], osems[b]
        ).wait()


def kernel(x, table):
    B, H = x.shape
    V, D = table.shape
    per_w = B // NW
    assert per_w * NW == B and per_w % BT == 0 and D == 64 and BT == 128

    mesh = plsc.VectorSubcoreMesh(core_axis_name="c", subcore_axis_name="s")
    run = pl.kernel(
        functools.partial(_gather_kernel, per_w, H, D),
        out_type=jax.ShapeDtypeStruct((H, D // 8, B // BT, 8, BT), jnp.float32),
        mesh=mesh,
        scratch_types=[
            pltpu.VMEM((per_w, H), jnp.int32),
            pltpu.VMEM((H, per_w), jnp.int32),
            pltpu.VMEM((NBUF, BT, D), jnp.float32),
            pltpu.VMEM((BT, 65), jnp.float32),
            pltpu.VMEM((NBUF, D // 8, 8, BT), jnp.float32),
            [pltpu.SemaphoreType.DMA] * NBUF,
            [pltpu.SemaphoreType.DMA] * NBUF,
        ],
        compiler_params=pltpu.CompilerParams(
            use_tc_tiling_on_sc=False, needs_layout_passes=False),
    )
    out5 = run(x.astype(jnp.int32), table)
    return out5.transpose(2, 4, 0, 1, 3).reshape(B, H, D)


# speedup vs baseline: 1.5584x; 1.0617x over previous
import functools

import jax
import jax.numpy as jnp
from jax import lax
from jax.experimental import pallas as pl
from jax.experimental.pallas import tpu as pltpu
from jax.experimental.pallas import tpu_sc as plsc

NW = 32
NBUF = 8


def _gather_kernel(per_w, x_hbm, table_hbm, out_hbm,
                   idx_v, rows_v, gsems, osems):
    nc = 2
    wid = lax.axis_index("s") * nc + lax.axis_index("c")
    s0 = wid * per_w
    pltpu.sync_copy(x_hbm.at[pl.ds(s0, per_w)], idx_v)

    @pl.loop(0, per_w, step=NBUF)
    def group(i0):
        descs = []
        for b in range(NBUF):
            @pl.when(i0 > 0)
            def _(b=b):
                pltpu.make_async_copy(
                    rows_v.at[b], out_hbm.at[0], osems[b]
                ).wait()
            descs.append(
                pltpu.async_copy(
                    table_hbm.at[idx_v.at[i0 + b]], rows_v.at[b], gsems[b]
                )
            )
        for b in range(NBUF):
            descs[b].wait()
            pltpu.async_copy(
                rows_v.at[b], out_hbm.at[s0 + i0 + b], osems[b]
            )

    for b in range(NBUF):
        pltpu.make_async_copy(
            rows_v.at[b], out_hbm.at[0], osems[b]
        ).wait()


def kernel(x, table):
    B, H = x.shape
    V, D = table.shape
    per_w = B // NW
    assert per_w * NW == B and per_w % NBUF == 0

    mesh = plsc.VectorSubcoreMesh(core_axis_name="c", subcore_axis_name="s")
    run = pl.kernel(
        functools.partial(_gather_kernel, per_w),
        out_type=jax.ShapeDtypeStruct((B, H, D), jnp.float32),
        mesh=mesh,
        scratch_types=[
            pltpu.VMEM((per_w, H), jnp.int32),
            pltpu.VMEM((NBUF, H, D), jnp.float32),
            [pltpu.SemaphoreType.DMA] * NBUF,
            [pltpu.SemaphoreType.DMA] * NBUF,
        ],
        compiler_params=pltpu.CompilerParams(use_tc_tiling_on_sc=False),
    )
    return run(x.astype(jnp.int32), table)
